# speedup vs baseline: 458.9810x; 458.9810x over previous
import jax
import jax.numpy as jnp
from jax import lax
from jax.experimental import pallas as pl


def _gnn_row_kernel(x_ref, lw_ref, sw_ref, dw_ref, bw_ref, fw_ref, o_ref):
    x = x_ref[0, 0]
    n_layers = lw_ref.shape[1]
    for l in range(n_layers):
        w = lw_ref[0, l]
        sv = sw_ref[0, l:l + 1, :]
        dv = dw_ref[0, l:l + 1, :]
        bv = bw_ref[0, l:l + 1, :]
        h = lax.dot_general(x, w, (((1,), (1,)), ((), ())),
                            preferred_element_type=jnp.float32)
        a_src = lax.dot_general(sv, h, (((1,), (1,)), ((), ())),
                                preferred_element_type=jnp.float32)
        a_dst = lax.dot_general(h, dv, (((1,), (1,)), ((), ())),
                                preferred_element_type=jnp.float32)
        logits = a_src + a_dst
        logits = jnp.where(logits >= 0, logits, 0.2 * logits)
        m = jnp.max(logits, axis=1, keepdims=True)
        ex = jnp.exp(logits - m)
        s = jnp.sum(ex, axis=1, keepdims=True)
        alpha = ex / (s + 1e-16)
        x = jnp.dot(alpha, h, preferred_element_type=jnp.float32) + bv
    rowsum = jnp.sum(x, axis=0, keepdims=True)
    y = lax.dot_general(rowsum, fw_ref[0], (((1,), (1,)), ((), ())),
                        preferred_element_type=jnp.float32)
    o_ref[0, 0] = y


def kernel(xs, pos_enc, gat_lin_w, gat_src_w, gat_dst_w, gat_bias_w, lin_w):
    bs, num_rows, num_xs = xs.shape
    enc = pos_enc.shape[-1]
    hid = gat_lin_w.shape[-1]
    out_dim = lin_w.shape[-2]
    pe = jnp.broadcast_to(pos_enc[:, None, :, :], (bs, num_rows, num_xs, enc))
    x0 = jnp.concatenate([xs[..., None], pe], axis=-1)

    grid = (bs, num_rows)
    out = pl.pallas_call(
        _gnn_row_kernel,
        grid=grid,
        in_specs=[
            pl.BlockSpec((1, 1, num_xs, hid), lambda b, r: (b, r, 0, 0)),
            pl.BlockSpec((1,) + gat_lin_w.shape[1:], lambda b, r: (b, 0, 0, 0)),
            pl.BlockSpec((1,) + gat_src_w.shape[1:], lambda b, r: (b, 0, 0)),
            pl.BlockSpec((1,) + gat_dst_w.shape[1:], lambda b, r: (b, 0, 0)),
            pl.BlockSpec((1,) + gat_bias_w.shape[1:], lambda b, r: (b, 0, 0)),
            pl.BlockSpec((1,) + lin_w.shape[1:], lambda b, r: (b, 0, 0)),
        ],
        out_specs=pl.BlockSpec((1, 1, 1, out_dim), lambda b, r: (b, r, 0, 0)),
        out_shape=jax.ShapeDtypeStruct((bs, num_rows, 1, out_dim), jnp.float32),
    )(x0, gat_lin_w, gat_src_w, gat_dst_w, gat_bias_w, lin_w)
    return out.reshape(bs, num_rows, out_dim)
